# Initial kernel scaffold; baseline (speedup 1.0000x reference)
import jax
import jax.numpy as jnp
from jax.experimental import pallas as pl


def kernel(x):
    raise NotImplementedError("write your pallas kernel here")


# speedup vs baseline: 1.9950x; 1.9950x over previous
import functools

import jax
import jax.numpy as jnp
import numpy as np
from jax import lax
from jax.experimental import pallas as pl
from jax.experimental.pallas import tpu as pltpu
from jax.experimental.pallas import tpu_sc as plsc

_EMB = 32
_LEN = 512
_LB = -15.0
_UB = 16.0
_LN_BASE = float(np.log(256.0 ** (1.0 / 15.0)))

_TAB = 32768
_ROWS = 4 * _LEN
_ROW = _LEN * _EMB


def _table_body(out_ref):
    r = lax.broadcasted_iota(jnp.int32, (256, 128), 0)
    c = lax.broadcasted_iota(jnp.int32, (256, 128), 1)
    i = r * 128 + c
    d = (i >> 5) - (_LEN - 1)
    e = i & (_EMB - 1)
    sign = jnp.sign(d).astype(jnp.float32)
    a = jnp.abs(d).astype(jnp.float32)
    v = jnp.floor(jnp.log(a) / _LN_BASE + 1.0)
    v = jnp.where(v < 0, 0.0, v)
    v = v * sign
    v = jnp.where(v < _LB, _LB, v)
    v = jnp.where(v > _UB, _UB, v)
    idx = (v - _LB).astype(jnp.int32)
    out_ref[...] = (idx == e).astype(jnp.float32)


_build_table = pl.pallas_call(
    _table_body,
    out_shape=jax.ShapeDtypeStruct((_TAB // 128, 128), jnp.float32),
)

_NUM_CORES = 2
_NW = _NUM_CORES * 16
_RPW = _ROWS // _NW


def _fanout_body(m_hbm, out_hbm, m_v):
    wid = lax.axis_index("s") * _NUM_CORES + lax.axis_index("c")
    pltpu.sync_copy(m_hbm, m_v)

    def body(i, carry):
        t = wid * _RPW + i
        j = lax.rem(t, _LEN)
        off = (_LEN - 1 - j) * _EMB
        pltpu.sync_copy(m_v.at[pl.ds(off, _ROW)], out_hbm.at[pl.ds(t * _ROW, _ROW)])
        return carry

    lax.fori_loop(0, _RPW, body, 0)


@functools.cache
def _get_fanout():
    return functools.partial(
        pl.kernel,
        out_type=jax.ShapeDtypeStruct((_ROWS * _ROW,), jnp.float32),
        mesh=plsc.VectorSubcoreMesh(
            core_axis_name="c",
            subcore_axis_name="s",
            num_cores=_NUM_CORES,
            num_subcores=16,
        ),
        scratch_types=[pltpu.VMEM((_TAB,), jnp.float32)],
    )(_fanout_body)


def kernel(x):
    batch, length = x.shape
    m = _build_table().reshape(_TAB)
    out = _get_fanout()(m)
    return out.reshape(batch, length, length, _EMB)


# speedup vs baseline: 1.9996x; 1.0023x over previous
import functools

import jax
import jax.numpy as jnp
import numpy as np
from jax import lax
from jax.experimental import pallas as pl
from jax.experimental.pallas import tpu as pltpu
from jax.experimental.pallas import tpu_sc as plsc

_EMB = 32
_LEN = 512
_LB = -15.0
_UB = 16.0
_LN_BASE = float(np.log(256.0 ** (1.0 / 15.0)))

_TAB = 32768
_ROWS = 4 * _LEN
_ROW = _LEN * _EMB


def _table_body(out_ref):
    r = lax.broadcasted_iota(jnp.int32, (256, 128), 0)
    c = lax.broadcasted_iota(jnp.int32, (256, 128), 1)
    i = r * 128 + c
    d = (i >> 5) - (_LEN - 1)
    e = i & (_EMB - 1)
    sign = jnp.sign(d).astype(jnp.float32)
    a = jnp.abs(d).astype(jnp.float32)
    v = jnp.floor(jnp.log(a) / _LN_BASE + 1.0)
    v = jnp.where(v < 0, 0.0, v)
    v = v * sign
    v = jnp.where(v < _LB, _LB, v)
    v = jnp.where(v > _UB, _UB, v)
    idx = (v - _LB).astype(jnp.int32)
    out_ref[...] = (idx == e).astype(jnp.float32)


_build_table = pl.pallas_call(
    _table_body,
    out_shape=jax.ShapeDtypeStruct((_TAB // 128, 128), jnp.float32),
)

_NUM_CORES = 2
_NW = _NUM_CORES * 16
_RPW = _ROWS // _NW


def _fanout_body(m_hbm, out_hbm, m_v, sem):
    wid = lax.axis_index("s") * _NUM_CORES + lax.axis_index("c")
    pltpu.sync_copy(m_hbm, m_v)

    def fire(i, carry):
        t = wid * _RPW + i
        j = lax.rem(t, _LEN)
        off = (_LEN - 1 - j) * _EMB
        pltpu.async_copy(m_v.at[pl.ds(off, _ROW)], out_hbm.at[pl.ds(t * _ROW, _ROW)], sem)
        return carry

    lax.fori_loop(0, _RPW, fire, 0)

    def drain(i, carry):
        pltpu.make_async_copy(
            m_v.at[pl.ds(0, _ROW)], out_hbm.at[pl.ds(0, _ROW)], sem
        ).wait()
        return carry

    lax.fori_loop(0, _RPW, drain, 0)


@functools.cache
def _get_fanout():
    return functools.partial(
        pl.kernel,
        out_type=jax.ShapeDtypeStruct((_ROWS * _ROW,), jnp.float32),
        mesh=plsc.VectorSubcoreMesh(
            core_axis_name="c",
            subcore_axis_name="s",
            num_cores=_NUM_CORES,
            num_subcores=16,
        ),
        scratch_types=[pltpu.VMEM((_TAB,), jnp.float32), pltpu.SemaphoreType.DMA],
    )(_fanout_body)


def kernel(x):
    batch, length = x.shape
    m = _build_table().reshape(_TAB)
    out = _get_fanout()(m)
    return out.reshape(batch, length, length, _EMB)


# speedup vs baseline: 12.0577x; 6.0302x over previous
import functools

import jax
import jax.numpy as jnp
import numpy as np
from jax import lax
from jax.experimental import pallas as pl
from jax.experimental.pallas import tpu as pltpu

_EMB = 32
_LEN = 512
_LB = -15.0
_UB = 16.0
_LN_BASE = float(np.log(256.0 ** (1.0 / 15.0)))

_BJ = 16


def _bucket(d):
    sign = jnp.sign(d).astype(jnp.float32)
    a = jnp.abs(d).astype(jnp.float32)
    v = jnp.floor(jnp.log(a) / _LN_BASE + 1.0)
    v = jnp.where(v < 0, 0.0, v)
    v = v * sign
    v = jnp.where(v < _LB, _LB, v)
    v = jnp.where(v > _UB, _UB, v)
    return (v - _LB).astype(jnp.int32)


def _plane_body(o_ref, lohi_ref):
    b = pl.program_id(0)
    jb = pl.program_id(1)

    @pl.when(jnp.logical_and(b == 0, jb == 0))
    def _():
        dd = lax.broadcasted_iota(jnp.int32, (_EMB, 1024), 1) - (_LEN - 1)
        e = lax.broadcasted_iota(jnp.int32, (_EMB, 1024), 0)
        m = _bucket(dd) == e
        dfl = dd.astype(jnp.float32)
        lohi_ref[:, 0:1] = jnp.min(jnp.where(m, dfl, 1e9), axis=1, keepdims=True)
        lohi_ref[:, 1:2] = jnp.max(jnp.where(m, dfl, -1e9), axis=1, keepdims=True)

    lo = lohi_ref[:, 0:1].reshape(1, 1, _EMB, 1)
    hi = lohi_ref[:, 1:2].reshape(1, 1, _EMB, 1)
    kk = lax.broadcasted_iota(jnp.int32, (1, _BJ, _EMB, _LEN), 3)
    jj = lax.broadcasted_iota(jnp.int32, (1, _BJ, _EMB, _LEN), 1)
    d = (kk - jj - jb * _BJ).astype(jnp.float32)
    v = jnp.minimum(d - lo + 1.0, hi - d + 1.0)
    o_ref[...] = jnp.clip(v, 0.0, 1.0)


_planes = pl.pallas_call(
    _plane_body,
    grid=(4, _LEN // _BJ),
    out_specs=pl.BlockSpec((1, _BJ, _EMB, _LEN), lambda b, j: (b, j, 0, 0)),
    out_shape=jax.ShapeDtypeStruct((4, _LEN, _EMB, _LEN), jnp.float32),
    scratch_shapes=[pltpu.VMEM((_EMB, 128), jnp.float32)],
)


def kernel(x):
    batch, length = x.shape
    out = _planes()
    return jnp.transpose(out, (0, 1, 3, 2))


# speedup vs baseline: 25.3926x; 2.1059x over previous
import functools

import jax
import jax.numpy as jnp
import numpy as np
from jax import lax
from jax.experimental import pallas as pl
from jax.experimental.pallas import tpu as pltpu

_EMB = 32
_LEN = 512
_LB = -15.0
_UB = 16.0
_LN_BASE = float(np.log(256.0 ** (1.0 / 15.0)))

_BJ = 16


def _bucket(d):
    sign = jnp.sign(d).astype(jnp.float32)
    a = jnp.abs(d).astype(jnp.float32)
    v = jnp.floor(jnp.log(a) / _LN_BASE + 1.0)
    v = jnp.where(v < 0, 0.0, v)
    v = v * sign
    v = jnp.where(v < _LB, _LB, v)
    v = jnp.where(v > _UB, _UB, v)
    return (v - _LB).astype(jnp.int32)


def _plane_body(o_ref, lohi_ref):
    jb = pl.program_id(0)

    @pl.when(jb == 0)
    def _():
        dd = lax.broadcasted_iota(jnp.int32, (_EMB, 1024), 1) - (_LEN - 1)
        e = lax.broadcasted_iota(jnp.int32, (_EMB, 1024), 0)
        m = _bucket(dd) == e
        dfl = dd.astype(jnp.float32)
        lohi_ref[:, 0:1] = jnp.min(jnp.where(m, dfl, 1e9), axis=1, keepdims=True)
        lohi_ref[:, 1:2] = jnp.max(jnp.where(m, dfl, -1e9), axis=1, keepdims=True)

    lo = lohi_ref[:, 0:1].reshape(1, 1, _EMB, 1)
    hi = lohi_ref[:, 1:2].reshape(1, 1, _EMB, 1)
    kk = lax.broadcasted_iota(jnp.int32, (1, _BJ, _EMB, _LEN), 3)
    jj = lax.broadcasted_iota(jnp.int32, (1, _BJ, _EMB, _LEN), 1)
    d = (kk - jj - jb * _BJ).astype(jnp.float32)
    v = jnp.clip(jnp.minimum(d - lo + 1.0, hi - d + 1.0), 0.0, 1.0)
    o_ref[...] = jnp.broadcast_to(v, (4, _BJ, _EMB, _LEN))


_planes = pl.pallas_call(
    _plane_body,
    grid=(_LEN // _BJ,),
    out_specs=pl.BlockSpec((4, _BJ, _EMB, _LEN), lambda j: (0, j, 0, 0)),
    out_shape=jax.ShapeDtypeStruct((4, _LEN, _EMB, _LEN), jnp.float32),
    scratch_shapes=[pltpu.VMEM((_EMB, 128), jnp.float32)],
)


def kernel(x):
    batch, length = x.shape
    out = _planes()
    return jnp.transpose(out, (0, 1, 3, 2))
